# Initial kernel scaffold; baseline (speedup 1.0000x reference)
import jax
import jax.numpy as jnp
from jax.experimental import pallas as pl


def kernel(z, pos, edge_index, emb, msg_w1, msg_b1, msg_w2, msg_b2, upd_w1, upd_b1, upd_w2, upd_b2, out_w1, out_b1, out_w2, out_b2):
    raise NotImplementedError("write your pallas kernel here")


# speedup vs baseline: 2.3813x; 2.3813x over previous
import functools

import jax
import jax.numpy as jnp
from jax import lax
from jax.experimental import pallas as pl
from jax.experimental.pallas import tpu as pltpu
from jax.experimental.pallas import tpu_sc as plsc

HID = 128
RBF = 32
NL = 4
CUTOFF = 10.0
GAMMA = RBF / CUTOFF
NC = 2
NS = 16
NW = NC * NS
C = 64
RN = 1024


def _mesh():
    return plsc.VectorSubcoreMesh(core_axis_name="c", subcore_axis_name="s",
                                  num_cores=NC, num_subcores=NS)


@functools.lru_cache(maxsize=None)
def _sc_prep(Np, Ep):
    EPW = Ep // NW
    nchunk = EPW // C
    RZ = Np // NS

    @functools.partial(
        pl.kernel,
        out_type=(jax.ShapeDtypeStruct((Ep, 8), jnp.float32),
                  jax.ShapeDtypeStruct((Ep, 8), jnp.float32),
                  jax.ShapeDtypeStruct((NC, Np, 16), jnp.float32)),
        mesh=_mesh(),
        scratch_types=[
            pltpu.VMEM((C,), jnp.int32), pltpu.VMEM((C,), jnp.int32),
            pltpu.VMEM((C, 8), jnp.float32), pltpu.VMEM((C, 8), jnp.float32),
            pltpu.VMEM((C, 16), jnp.float32),
            pltpu.VMEM_SHARED((Np, 16), jnp.float32),
            pltpu.SemaphoreType.DMA, pltpu.SemaphoreType.DMA,
        ],
        compiler_params=pltpu.CompilerParams(use_tc_tiling_on_sc=False),
    )
    def k(pos_hbm, ii_hbm, jj_hbm, pi_out, pj_out, dg_out,
          ii_v, jj_v, pi_v, pj_v, ov, d_sh, s1, s2):
        cid = lax.axis_index("c")
        sid = lax.axis_index("s")
        w = sid * NC + cid
        base0 = w * EPW
        zero16 = jnp.zeros((16,), jnp.float32)

        def zloop(r, carry):
            ov[r, pl.ds(0, 16)] = zero16
            return carry

        lax.fori_loop(0, C, zloop, 0)
        for t in range(RZ // C):
            pltpu.sync_copy(ov, d_sh.at[pl.ds(sid * RZ + t * C, C)])
        plsc.subcore_barrier()

        dvec = jnp.where(lax.iota(jnp.int32, 16) == 0, 1.0, 0.0).astype(
            jnp.float32)

        def dloop(r, carry):
            ov[r, pl.ds(0, 16)] = dvec
            return carry

        lax.fori_loop(0, C, dloop, 0)

        def chunk(t, carry):
            b = base0 + t * C
            pltpu.sync_copy(ii_hbm.at[pl.ds(b, C)], ii_v)
            pltpu.sync_copy(jj_hbm.at[pl.ds(b, C)], jj_v)
            c1 = pltpu.async_copy(pos_hbm.at[ii_v], pi_v, s1)
            c2 = pltpu.async_copy(pos_hbm.at[jj_v], pj_v, s2)
            c1.wait()
            c2.wait()
            pltpu.sync_copy(pi_v, pi_out.at[pl.ds(b, C)])
            pltpu.sync_copy(pj_v, pj_out.at[pl.ds(b, C)])
            pltpu.sync_copy(ov, d_sh.at[ii_v], add=True)
            return carry

        lax.fori_loop(0, nchunk, chunk, 0)
        plsc.subcore_barrier()
        for t in range(RZ // C):
            off = sid * RZ + t * C
            pltpu.sync_copy(d_sh.at[pl.ds(off, C)], dg_out.at[cid, pl.ds(off, C)])

    return k


@functools.lru_cache(maxsize=None)
def _sc_edge(Np, Ep):
    EPW = Ep // NW
    nchunk = EPW // C
    RZ = Np // NS

    @functools.partial(
        pl.kernel,
        out_type=jax.ShapeDtypeStruct((NC, Np, HID), jnp.float32),
        mesh=_mesh(),
        scratch_types=[
            pltpu.VMEM((C,), jnp.int32), pltpu.VMEM((C,), jnp.int32),
            pltpu.VMEM((C, HID), jnp.float32), pltpu.VMEM((C, HID), jnp.float32),
            pltpu.VMEM((C, HID), jnp.float32), pltpu.VMEM((C, HID), jnp.float32),
            pltpu.VMEM_SHARED((Np, HID), jnp.float32),
            pltpu.SemaphoreType.DMA, pltpu.SemaphoreType.DMA,
            pltpu.SemaphoreType.DMA,
        ],
    )
    def k(a_hbm, b_hbm, ii_hbm, jj_hbm, rp_hbm, s_out,
          ii_v, jj_v, av, bv, rv, ov, s_sh, m1, m2, m3):
        cid = lax.axis_index("c")
        sid = lax.axis_index("s")
        w = sid * NC + cid
        base0 = w * EPW
        zero16 = jnp.zeros((16,), jnp.float32)

        def zloop(f, carry):
            r = f // (HID // 16)
            g = (f % (HID // 16)) * 16
            ov[r, pl.ds(g, 16)] = zero16
            return carry

        lax.fori_loop(0, C * (HID // 16), zloop, 0)
        for t in range(RZ // C):
            pltpu.sync_copy(ov, s_sh.at[pl.ds(sid * RZ + t * C, C)])
        plsc.subcore_barrier()

        def chunk(t, carry):
            b = base0 + t * C
            pltpu.sync_copy(ii_hbm.at[pl.ds(b, C)], ii_v)
            pltpu.sync_copy(jj_hbm.at[pl.ds(b, C)], jj_v)
            c1 = pltpu.async_copy(a_hbm.at[ii_v], av, m1)
            c2 = pltpu.async_copy(b_hbm.at[jj_v], bv, m2)
            c3 = pltpu.async_copy(rp_hbm.at[pl.ds(b, C)], rv, m3)
            c1.wait()
            c2.wait()
            c3.wait()

            def comp(r, cc):
                for g in range(HID // 16):
                    o = g * 16
                    ov[r, pl.ds(o, 16)] = jnp.maximum(
                        av[r, pl.ds(o, 16)] + bv[r, pl.ds(o, 16)]
                        + rv[r, pl.ds(o, 16)], 0.0)
                return cc

            lax.fori_loop(0, C, comp, 0)
            pltpu.sync_copy(ov, s_sh.at[ii_v], add=True)
            return carry

        lax.fori_loop(0, nchunk, chunk, 0)
        plsc.subcore_barrier()
        for t in range(RZ // C):
            off = sid * RZ + t * C
            pltpu.sync_copy(s_sh.at[pl.ds(off, C)], s_out.at[cid, pl.ds(off, C)])

    return k


@functools.lru_cache(maxsize=None)
def _tc_rp(Ep):
    Rb = 2048

    def body(pi_ref, pj_ref, w_ref, b_ref, out_ref):
        diff = pi_ref[...] - pj_ref[...]
        d = jnp.sqrt(jnp.sum(diff * diff, axis=1, keepdims=True))
        centers = (CUTOFF / (RBF - 1)) * lax.broadcasted_iota(
            jnp.int32, (1, RBF), 1).astype(jnp.float32)
        rbf = jnp.exp(-GAMMA * (d - centers) ** 2)
        out_ref[0] = jnp.dot(rbf, w_ref[0],
                             preferred_element_type=jnp.float32) + b_ref[0]

    return pl.pallas_call(
        body,
        grid=(NL, Ep // Rb),
        in_specs=[
            pl.BlockSpec((Rb, 8), lambda l, e: (e, 0)),
            pl.BlockSpec((Rb, 8), lambda l, e: (e, 0)),
            pl.BlockSpec((1, RBF, HID), lambda l, e: (l, 0, 0)),
            pl.BlockSpec((1, 1, HID), lambda l, e: (l, 0, 0)),
        ],
        out_specs=pl.BlockSpec((1, Rb, HID), lambda l, e: (l, e, 0)),
        out_shape=jax.ShapeDtypeStruct((NL, Ep, HID), jnp.float32),
    )


@functools.lru_cache(maxsize=None)
def _tc_embed(Np):
    def body(z_ref, e_ref, wi_ref, wj_ref, h_ref, a_ref, b_ref):
        ids = lax.broadcasted_iota(jnp.int32, (1, 16), 1).astype(jnp.float32)
        oh = (z_ref[...] == ids).astype(jnp.float32)
        h = jnp.dot(oh, e_ref[...], preferred_element_type=jnp.float32)
        h_ref[...] = h
        a_ref[...] = jnp.dot(h, wi_ref[...], preferred_element_type=jnp.float32)
        b_ref[...] = jnp.dot(h, wj_ref[...], preferred_element_type=jnp.float32)

    sds = jax.ShapeDtypeStruct((Np, HID), jnp.float32)
    return pl.pallas_call(
        body,
        grid=(Np // RN,),
        in_specs=[
            pl.BlockSpec((RN, 1), lambda i: (i, 0)),
            pl.BlockSpec((16, HID), lambda i: (0, 0)),
            pl.BlockSpec((HID, HID), lambda i: (0, 0)),
            pl.BlockSpec((HID, HID), lambda i: (0, 0)),
        ],
        out_specs=[pl.BlockSpec((RN, HID), lambda i: (i, 0))] * 3,
        out_shape=[sds, sds, sds],
    )


def _node_update(s_ref, dg_ref, h_ref, w2_ref, b2_ref, u1a_ref, u1b_ref,
                 c1_ref, u2_ref, c2_ref):
    ssum = s_ref[0] + s_ref[1]
    deg = dg_ref[0, :, :1] + dg_ref[1, :, :1]
    aggr = (jnp.dot(ssum, w2_ref[...], preferred_element_type=jnp.float32)
            + deg * b2_ref[...]) / jnp.maximum(deg, 1.0)
    u = jnp.maximum(
        jnp.dot(h_ref[...], u1a_ref[...], preferred_element_type=jnp.float32)
        + jnp.dot(aggr, u1b_ref[...], preferred_element_type=jnp.float32)
        + c1_ref[...], 0.0)
    return jnp.dot(u, u2_ref[...], preferred_element_type=jnp.float32) + c2_ref[...]


_W = pl.BlockSpec((HID, HID), lambda i: (0, 0))
_B = pl.BlockSpec((1, HID), lambda i: (0, 0))
_S = pl.BlockSpec((NC, RN, HID), lambda i: (0, i, 0))
_D = pl.BlockSpec((NC, RN, 16), lambda i: (0, i, 0))
_H = pl.BlockSpec((RN, HID), lambda i: (i, 0))


@functools.lru_cache(maxsize=None)
def _tc_update(Np):
    def body(s_ref, dg_ref, h_ref, w2_ref, b2_ref, u1a_ref, u1b_ref, c1_ref,
             u2_ref, c2_ref, wi_ref, wj_ref, hn_ref, an_ref, bn_ref):
        hn = _node_update(s_ref, dg_ref, h_ref, w2_ref, b2_ref, u1a_ref,
                          u1b_ref, c1_ref, u2_ref, c2_ref)
        hn_ref[...] = hn
        an_ref[...] = jnp.dot(hn, wi_ref[...], preferred_element_type=jnp.float32)
        bn_ref[...] = jnp.dot(hn, wj_ref[...], preferred_element_type=jnp.float32)

    sds = jax.ShapeDtypeStruct((Np, HID), jnp.float32)
    return pl.pallas_call(
        body,
        grid=(Np // RN,),
        in_specs=[_S, _D, _H, _W, _B, _W, _W, _B, _W, _B, _W, _W],
        out_specs=[_H] * 3,
        out_shape=[sds, sds, sds],
    )


@functools.lru_cache(maxsize=None)
def _tc_final(Np, N):
    def body(s_ref, dg_ref, h_ref, w2_ref, b2_ref, u1a_ref, u1b_ref, c1_ref,
             u2_ref, c2_ref, ow1_ref, ob1_ref, ow2_ref, ob2_ref, out_ref):
        pid = pl.program_id(0)
        hn = _node_update(s_ref, dg_ref, h_ref, w2_ref, b2_ref, u1a_ref,
                          u1b_ref, c1_ref, u2_ref, c2_ref)
        hid = jnp.maximum(
            jnp.dot(hn, ow1_ref[...], preferred_element_type=jnp.float32)
            + ob1_ref[...], 0.0)
        eatom = (jnp.dot(hid, ow2_ref[...], preferred_element_type=jnp.float32)
                 + ob2_ref[...])
        rid = pid * RN + lax.broadcasted_iota(jnp.int32, (RN, 1), 0)
        blk = jnp.full((1, HID), jnp.sum(jnp.where(rid < N, eatom, 0.0)))

        @pl.when(pid == 0)
        def _():
            out_ref[...] = blk

        @pl.when(pid != 0)
        def _():
            out_ref[...] += blk

    return pl.pallas_call(
        body,
        grid=(Np // RN,),
        in_specs=[
            _S, _D, _H, _W, _B, _W, _W, _B, _W, _B,
            _W, _B,
            pl.BlockSpec((HID, 1), lambda i: (0, 0)),
            pl.BlockSpec((1, 1), lambda i: (0, 0)),
        ],
        out_specs=pl.BlockSpec((1, HID), lambda i: (0, 0)),
        out_shape=jax.ShapeDtypeStruct((1, HID), jnp.float32),
    )


def kernel(z, pos, edge_index, emb, msg_w1, msg_b1, msg_w2, msg_b2,
           upd_w1, upd_b1, upd_w2, upd_b2, out_w1, out_b1, out_w2, out_b2):
    N = z.shape[0]
    E = edge_index.shape[1]
    Np = -(-N // (NS * C)) * (NS * C)
    Ep = -(-E // (NW * C)) * (NW * C)

    zf = jnp.pad(z.astype(jnp.float32), (0, Np - N)).reshape(Np, 1)
    pos8 = jnp.pad(pos, ((0, Np - N), (0, 5)))
    ii = jnp.pad(edge_index[0].astype(jnp.int32), (0, Ep - E),
                 constant_values=Np - 1)
    jj = jnp.pad(edge_index[1].astype(jnp.int32), (0, Ep - E),
                 constant_values=Np - 1)
    emb16 = jnp.pad(emb, ((0, 16 - emb.shape[0]), (0, 0)))
    w1i = msg_w1[:, :HID, :]
    w1j = msg_w1[:, HID:2 * HID, :]
    w1r = msg_w1[:, 2 * HID:, :]

    pi8, pj8, dg = _sc_prep(Np, Ep)(pos8, ii, jj)
    rp = _tc_rp(Ep)(pi8, pj8, w1r, msg_b1.reshape(NL, 1, HID))
    h, a, b = _tc_embed(Np)(zf, emb16, w1i[0], w1j[0])

    for l in range(NL - 1):
        s2 = _sc_edge(Np, Ep)(a, b, ii, jj, rp[l])
        h, a, b = _tc_update(Np)(
            s2, dg, h, msg_w2[l], msg_b2[l].reshape(1, HID),
            upd_w1[l, :HID], upd_w1[l, HID:], upd_b1[l].reshape(1, HID),
            upd_w2[l], upd_b2[l].reshape(1, HID), w1i[l + 1], w1j[l + 1])

    s2 = _sc_edge(Np, Ep)(a, b, ii, jj, rp[NL - 1])
    energy = _tc_final(Np, N)(
        s2, dg, h, msg_w2[NL - 1], msg_b2[NL - 1].reshape(1, HID),
        upd_w1[NL - 1, :HID], upd_w1[NL - 1, HID:],
        upd_b1[NL - 1].reshape(1, HID), upd_w2[NL - 1],
        upd_b2[NL - 1].reshape(1, HID), out_w1, out_b1.reshape(1, HID),
        out_w2, out_b2.reshape(1, 1))
    return energy[0, 0]


# speedup vs baseline: 2.7436x; 1.1521x over previous
import functools

import jax
import jax.numpy as jnp
from jax import lax
from jax.experimental import pallas as pl
from jax.experimental.pallas import tpu as pltpu
from jax.experimental.pallas import tpu_sc as plsc

HID = 128
RBF = 32
NL = 4
CUTOFF = 10.0
GAMMA = RBF / CUTOFF
NC = 2
NS = 16
NW = NC * NS
C = 64
RN = 1024


def _mesh():
    return plsc.VectorSubcoreMesh(core_axis_name="c", subcore_axis_name="s",
                                  num_cores=NC, num_subcores=NS)


@functools.lru_cache(maxsize=None)
def _sc_prep(Np, Ep):
    EPW = Ep // NW
    nchunk = EPW // C
    RZ = Np // NS

    @functools.partial(
        pl.kernel,
        out_type=(jax.ShapeDtypeStruct((Ep, 8), jnp.float32),
                  jax.ShapeDtypeStruct((Ep, 8), jnp.float32),
                  jax.ShapeDtypeStruct((NC, Np, 16), jnp.float32)),
        mesh=_mesh(),
        scratch_types=[
            pltpu.VMEM((C,), jnp.int32), pltpu.VMEM((C,), jnp.int32),
            pltpu.VMEM((C, 8), jnp.float32), pltpu.VMEM((C, 8), jnp.float32),
            pltpu.VMEM((C, 16), jnp.float32),
            pltpu.VMEM_SHARED((Np, 16), jnp.float32),
            pltpu.SemaphoreType.DMA, pltpu.SemaphoreType.DMA,
        ],
        compiler_params=pltpu.CompilerParams(use_tc_tiling_on_sc=False),
    )
    def k(pos_hbm, ii_hbm, jj_hbm, pi_out, pj_out, dg_out,
          ii_v, jj_v, pi_v, pj_v, ov, d_sh, s1, s2):
        cid = lax.axis_index("c")
        sid = lax.axis_index("s")
        w = sid * NC + cid
        base0 = w * EPW
        zero16 = jnp.zeros((16,), jnp.float32)

        def zloop(r, carry):
            ov[r, pl.ds(0, 16)] = zero16
            return carry

        lax.fori_loop(0, C, zloop, 0)
        for t in range(RZ // C):
            pltpu.sync_copy(ov, d_sh.at[pl.ds(sid * RZ + t * C, C)])
        plsc.subcore_barrier()

        dvec = jnp.where(lax.iota(jnp.int32, 16) == 0, 1.0, 0.0).astype(
            jnp.float32)

        def dloop(r, carry):
            ov[r, pl.ds(0, 16)] = dvec
            return carry

        lax.fori_loop(0, C, dloop, 0)

        def chunk(t, carry):
            b = base0 + t * C
            pltpu.sync_copy(ii_hbm.at[pl.ds(b, C)], ii_v)
            pltpu.sync_copy(jj_hbm.at[pl.ds(b, C)], jj_v)
            c1 = pltpu.async_copy(pos_hbm.at[ii_v], pi_v, s1)
            c2 = pltpu.async_copy(pos_hbm.at[jj_v], pj_v, s2)
            c1.wait()
            c2.wait()
            pltpu.sync_copy(pi_v, pi_out.at[pl.ds(b, C)])
            pltpu.sync_copy(pj_v, pj_out.at[pl.ds(b, C)])
            pltpu.sync_copy(ov, d_sh.at[ii_v], add=True)
            return carry

        lax.fori_loop(0, nchunk, chunk, 0)
        plsc.subcore_barrier()
        for t in range(RZ // C):
            off = sid * RZ + t * C
            pltpu.sync_copy(d_sh.at[pl.ds(off, C)], dg_out.at[cid, pl.ds(off, C)])

    return k


CE = 32


@functools.lru_cache(maxsize=None)
def _sc_edge(Np, Ep):
    EPW = Ep // NW
    nchunk = EPW // CE
    assert nchunk % 2 == 0
    RZ = Np // NS

    @functools.partial(
        pl.kernel,
        out_type=jax.ShapeDtypeStruct((NC, Np, HID), jnp.float32),
        mesh=_mesh(),
        scratch_types=[
            pltpu.VMEM((2, CE), jnp.int32), pltpu.VMEM((2, CE), jnp.int32),
            pltpu.VMEM((2, CE, HID), jnp.float32),
            pltpu.VMEM((2, CE, HID), jnp.float32),
            pltpu.VMEM((2, CE, HID), jnp.float32),
            pltpu.VMEM_SHARED((Np, HID), jnp.float32),
            pltpu.SemaphoreType.DMA, pltpu.SemaphoreType.DMA,
        ],
    )
    def k(a_hbm, b_hbm, ii_hbm, jj_hbm, rp_hbm, s_out,
          ii_v, jj_v, av, bv, rv, s_sh, g0, g1):
        cid = lax.axis_index("c")
        sid = lax.axis_index("s")
        w = sid * NC + cid
        base0 = w * EPW
        zero16 = jnp.zeros((16,), jnp.float32)
        sems = (g0, g1)

        def zloop(f, carry):
            r = f // (HID // 16)
            g = (f % (HID // 16)) * 16
            av[0, r, pl.ds(g, 16)] = zero16
            return carry

        lax.fori_loop(0, CE * (HID // 16), zloop, 0)
        for t in range(RZ // CE):
            pltpu.sync_copy(av.at[0], s_sh.at[pl.ds(sid * RZ + t * CE, CE)])
        plsc.subcore_barrier()

        def start(ph, kk):
            b = base0 + kk * CE
            pltpu.sync_copy(ii_hbm.at[pl.ds(b, CE)], ii_v.at[ph])
            pltpu.sync_copy(jj_hbm.at[pl.ds(b, CE)], jj_v.at[ph])
            pltpu.async_copy(a_hbm.at[ii_v.at[ph]], av.at[ph], sems[ph])
            pltpu.async_copy(b_hbm.at[jj_v.at[ph]], bv.at[ph], sems[ph])
            pltpu.async_copy(rp_hbm.at[pl.ds(b, CE)], rv.at[ph], sems[ph])

        def drain(ph):
            dummy = a_hbm.at[pl.ds(0, CE)]
            pltpu.make_async_copy(dummy, av.at[ph], sems[ph]).wait()
            pltpu.make_async_copy(dummy, bv.at[ph], sems[ph]).wait()
            pltpu.make_async_copy(dummy, rv.at[ph], sems[ph]).wait()

        start(0, 0)

        def body2(t, carry):
            for ph in (0, 1):
                kk = 2 * t + ph
                nxt = 1 - ph

                @pl.when(kk + 1 < nchunk)
                def _():
                    start(nxt, kk + 1)

                drain(ph)

                def comp(r, cc):
                    for g in range(HID // 16):
                        o = g * 16
                        av[ph, r, pl.ds(o, 16)] = jnp.maximum(
                            av[ph, r, pl.ds(o, 16)] + bv[ph, r, pl.ds(o, 16)]
                            + rv[ph, r, pl.ds(o, 16)], 0.0)
                    return cc

                lax.fori_loop(0, CE, comp, 0)
                pltpu.sync_copy(av.at[ph], s_sh.at[ii_v.at[ph]], add=True)
            return carry

        lax.fori_loop(0, nchunk // 2, body2, 0)
        plsc.subcore_barrier()
        for t in range(RZ // CE):
            off = sid * RZ + t * CE
            pltpu.sync_copy(s_sh.at[pl.ds(off, CE)], s_out.at[cid, pl.ds(off, CE)])

    return k


@functools.lru_cache(maxsize=None)
def _tc_rp(Ep):
    Rb = 2048

    def body(pi_ref, pj_ref, w_ref, b_ref, out_ref):
        diff = pi_ref[...] - pj_ref[...]
        d = jnp.sqrt(jnp.sum(diff * diff, axis=1, keepdims=True))
        centers = (CUTOFF / (RBF - 1)) * lax.broadcasted_iota(
            jnp.int32, (1, RBF), 1).astype(jnp.float32)
        rbf = jnp.exp(-GAMMA * (d - centers) ** 2)
        out_ref[0] = jnp.dot(rbf, w_ref[0],
                             preferred_element_type=jnp.float32) + b_ref[0]

    return pl.pallas_call(
        body,
        grid=(NL, Ep // Rb),
        in_specs=[
            pl.BlockSpec((Rb, 8), lambda l, e: (e, 0)),
            pl.BlockSpec((Rb, 8), lambda l, e: (e, 0)),
            pl.BlockSpec((1, RBF, HID), lambda l, e: (l, 0, 0)),
            pl.BlockSpec((1, 1, HID), lambda l, e: (l, 0, 0)),
        ],
        out_specs=pl.BlockSpec((1, Rb, HID), lambda l, e: (l, e, 0)),
        out_shape=jax.ShapeDtypeStruct((NL, Ep, HID), jnp.float32),
    )


@functools.lru_cache(maxsize=None)
def _tc_embed(Np):
    def body(z_ref, e_ref, wi_ref, wj_ref, h_ref, a_ref, b_ref):
        ids = lax.broadcasted_iota(jnp.int32, (1, 16), 1).astype(jnp.float32)
        oh = (z_ref[...] == ids).astype(jnp.float32)
        h = jnp.dot(oh, e_ref[...], preferred_element_type=jnp.float32)
        h_ref[...] = h
        a_ref[...] = jnp.dot(h, wi_ref[...], preferred_element_type=jnp.float32)
        b_ref[...] = jnp.dot(h, wj_ref[...], preferred_element_type=jnp.float32)

    sds = jax.ShapeDtypeStruct((Np, HID), jnp.float32)
    return pl.pallas_call(
        body,
        grid=(Np // RN,),
        in_specs=[
            pl.BlockSpec((RN, 1), lambda i: (i, 0)),
            pl.BlockSpec((16, HID), lambda i: (0, 0)),
            pl.BlockSpec((HID, HID), lambda i: (0, 0)),
            pl.BlockSpec((HID, HID), lambda i: (0, 0)),
        ],
        out_specs=[pl.BlockSpec((RN, HID), lambda i: (i, 0))] * 3,
        out_shape=[sds, sds, sds],
    )


def _node_update(s_ref, dg_ref, h_ref, w2_ref, b2_ref, u1a_ref, u1b_ref,
                 c1_ref, u2_ref, c2_ref):
    ssum = s_ref[0] + s_ref[1]
    deg = dg_ref[0, :, :1] + dg_ref[1, :, :1]
    aggr = (jnp.dot(ssum, w2_ref[...], preferred_element_type=jnp.float32)
            + deg * b2_ref[...]) / jnp.maximum(deg, 1.0)
    u = jnp.maximum(
        jnp.dot(h_ref[...], u1a_ref[...], preferred_element_type=jnp.float32)
        + jnp.dot(aggr, u1b_ref[...], preferred_element_type=jnp.float32)
        + c1_ref[...], 0.0)
    return jnp.dot(u, u2_ref[...], preferred_element_type=jnp.float32) + c2_ref[...]


_W = pl.BlockSpec((HID, HID), lambda i: (0, 0))
_B = pl.BlockSpec((1, HID), lambda i: (0, 0))
_S = pl.BlockSpec((NC, RN, HID), lambda i: (0, i, 0))
_D = pl.BlockSpec((NC, RN, 16), lambda i: (0, i, 0))
_H = pl.BlockSpec((RN, HID), lambda i: (i, 0))


@functools.lru_cache(maxsize=None)
def _tc_update(Np):
    def body(s_ref, dg_ref, h_ref, w2_ref, b2_ref, u1a_ref, u1b_ref, c1_ref,
             u2_ref, c2_ref, wi_ref, wj_ref, hn_ref, an_ref, bn_ref):
        hn = _node_update(s_ref, dg_ref, h_ref, w2_ref, b2_ref, u1a_ref,
                          u1b_ref, c1_ref, u2_ref, c2_ref)
        hn_ref[...] = hn
        an_ref[...] = jnp.dot(hn, wi_ref[...], preferred_element_type=jnp.float32)
        bn_ref[...] = jnp.dot(hn, wj_ref[...], preferred_element_type=jnp.float32)

    sds = jax.ShapeDtypeStruct((Np, HID), jnp.float32)
    return pl.pallas_call(
        body,
        grid=(Np // RN,),
        in_specs=[_S, _D, _H, _W, _B, _W, _W, _B, _W, _B, _W, _W],
        out_specs=[_H] * 3,
        out_shape=[sds, sds, sds],
    )


@functools.lru_cache(maxsize=None)
def _tc_final(Np, N):
    def body(s_ref, dg_ref, h_ref, w2_ref, b2_ref, u1a_ref, u1b_ref, c1_ref,
             u2_ref, c2_ref, ow1_ref, ob1_ref, ow2_ref, ob2_ref, out_ref):
        pid = pl.program_id(0)
        hn = _node_update(s_ref, dg_ref, h_ref, w2_ref, b2_ref, u1a_ref,
                          u1b_ref, c1_ref, u2_ref, c2_ref)
        hid = jnp.maximum(
            jnp.dot(hn, ow1_ref[...], preferred_element_type=jnp.float32)
            + ob1_ref[...], 0.0)
        eatom = (jnp.dot(hid, ow2_ref[...], preferred_element_type=jnp.float32)
                 + ob2_ref[...])
        rid = pid * RN + lax.broadcasted_iota(jnp.int32, (RN, 1), 0)
        blk = jnp.full((1, HID), jnp.sum(jnp.where(rid < N, eatom, 0.0)))

        @pl.when(pid == 0)
        def _():
            out_ref[...] = blk

        @pl.when(pid != 0)
        def _():
            out_ref[...] += blk

    return pl.pallas_call(
        body,
        grid=(Np // RN,),
        in_specs=[
            _S, _D, _H, _W, _B, _W, _W, _B, _W, _B,
            _W, _B,
            pl.BlockSpec((HID, 1), lambda i: (0, 0)),
            pl.BlockSpec((1, 1), lambda i: (0, 0)),
        ],
        out_specs=pl.BlockSpec((1, HID), lambda i: (0, 0)),
        out_shape=jax.ShapeDtypeStruct((1, HID), jnp.float32),
    )


def kernel(z, pos, edge_index, emb, msg_w1, msg_b1, msg_w2, msg_b2,
           upd_w1, upd_b1, upd_w2, upd_b2, out_w1, out_b1, out_w2, out_b2):
    N = z.shape[0]
    E = edge_index.shape[1]
    Np = -(-N // (NS * C)) * (NS * C)
    Ep = -(-E // (NW * C)) * (NW * C)

    zf = jnp.pad(z.astype(jnp.float32), (0, Np - N)).reshape(Np, 1)
    pos8 = jnp.pad(pos, ((0, Np - N), (0, 5)))
    ii = jnp.pad(edge_index[0].astype(jnp.int32), (0, Ep - E),
                 constant_values=Np - 1)
    jj = jnp.pad(edge_index[1].astype(jnp.int32), (0, Ep - E),
                 constant_values=Np - 1)
    emb16 = jnp.pad(emb, ((0, 16 - emb.shape[0]), (0, 0)))
    w1i = msg_w1[:, :HID, :]
    w1j = msg_w1[:, HID:2 * HID, :]
    w1r = msg_w1[:, 2 * HID:, :]

    pi8, pj8, dg = _sc_prep(Np, Ep)(pos8, ii, jj)
    rp = _tc_rp(Ep)(pi8, pj8, w1r, msg_b1.reshape(NL, 1, HID))
    h, a, b = _tc_embed(Np)(zf, emb16, w1i[0], w1j[0])

    for l in range(NL - 1):
        s2 = _sc_edge(Np, Ep)(a, b, ii, jj, rp[l])
        h, a, b = _tc_update(Np)(
            s2, dg, h, msg_w2[l], msg_b2[l].reshape(1, HID),
            upd_w1[l, :HID], upd_w1[l, HID:], upd_b1[l].reshape(1, HID),
            upd_w2[l], upd_b2[l].reshape(1, HID), w1i[l + 1], w1j[l + 1])

    s2 = _sc_edge(Np, Ep)(a, b, ii, jj, rp[NL - 1])
    energy = _tc_final(Np, N)(
        s2, dg, h, msg_w2[NL - 1], msg_b2[NL - 1].reshape(1, HID),
        upd_w1[NL - 1, :HID], upd_w1[NL - 1, HID:],
        upd_b1[NL - 1].reshape(1, HID), upd_w2[NL - 1],
        upd_b2[NL - 1].reshape(1, HID), out_w1, out_b1.reshape(1, HID),
        out_w2, out_b2.reshape(1, 1))
    return energy[0, 0]


# speedup vs baseline: 3.5823x; 1.3057x over previous
import functools

import jax
import jax.numpy as jnp
from jax import lax
from jax.experimental import pallas as pl
from jax.experimental.pallas import tpu as pltpu
from jax.experimental.pallas import tpu_sc as plsc

HID = 128
RBF = 32
NL = 4
CUTOFF = 10.0
GAMMA = RBF / CUTOFF
NC = 2
NS = 16
NW = NC * NS
C = 64
RN = 1024


def _mesh():
    return plsc.VectorSubcoreMesh(core_axis_name="c", subcore_axis_name="s",
                                  num_cores=NC, num_subcores=NS)


@functools.lru_cache(maxsize=None)
def _sc_prep(Np, Ep):
    EPW = Ep // NW
    nchunk = EPW // C
    RZ = Np // NS

    @functools.partial(
        pl.kernel,
        out_type=(jax.ShapeDtypeStruct((Ep, 8), jnp.float32),
                  jax.ShapeDtypeStruct((Ep, 8), jnp.float32),
                  jax.ShapeDtypeStruct((NC, Np, 16), jnp.float32)),
        mesh=_mesh(),
        scratch_types=[
            pltpu.VMEM((EPW // C, C), jnp.int32),
            pltpu.VMEM((EPW,), jnp.int32),
            pltpu.VMEM((C, 8), jnp.float32), pltpu.VMEM((C, 8), jnp.float32),
            pltpu.VMEM((C, 16), jnp.float32),
            pltpu.VMEM_SHARED((Np, 16), jnp.float32),
            pltpu.SemaphoreType.DMA, pltpu.SemaphoreType.DMA,
        ],
        compiler_params=pltpu.CompilerParams(use_tc_tiling_on_sc=False),
    )
    def k(pos_hbm, ii2_hbm, jj_hbm, pi_out, pj_out, dg_out,
          ii_loc, jj_loc, pi_v, pj_v, ov, d_sh, s1, s2):
        cid = lax.axis_index("c")
        sid = lax.axis_index("s")
        w = sid * NC + cid
        base0 = w * EPW
        zero16 = jnp.zeros((16,), jnp.float32)
        pltpu.sync_copy(ii2_hbm.at[w], ii_loc)
        pltpu.sync_copy(jj_hbm.at[pl.ds(base0, EPW)], jj_loc)

        def zloop(r, carry):
            ov[r, pl.ds(0, 16)] = zero16
            return carry

        lax.fori_loop(0, C, zloop, 0)
        for t in range(RZ // C):
            pltpu.sync_copy(ov, d_sh.at[pl.ds(sid * RZ + t * C, C)])
        plsc.subcore_barrier()

        dvec = jnp.where(lax.iota(jnp.int32, 16) == 0, 1.0, 0.0).astype(
            jnp.float32)

        def dloop(r, carry):
            ov[r, pl.ds(0, 16)] = dvec
            return carry

        lax.fori_loop(0, C, dloop, 0)

        def chunk(t, carry):
            b = base0 + t * C
            c1 = pltpu.async_copy(pos_hbm.at[ii_loc.at[t]], pi_v, s1)
            c2 = pltpu.async_copy(pos_hbm.at[jj_loc.at[pl.ds(t * C, C)]],
                                  pj_v, s2)
            c1.wait()
            c2.wait()
            pltpu.sync_copy(pi_v, pi_out.at[pl.ds(b, C)])
            pltpu.sync_copy(pj_v, pj_out.at[pl.ds(b, C)])
            pltpu.sync_copy(ov, d_sh.at[ii_loc.at[t]], add=True)
            return carry

        lax.fori_loop(0, nchunk, chunk, 0)
        plsc.subcore_barrier()
        for t in range(RZ // C):
            off = sid * RZ + t * C
            pltpu.sync_copy(d_sh.at[pl.ds(off, C)], dg_out.at[cid, pl.ds(off, C)])

    return k


CE = 32


@functools.lru_cache(maxsize=None)
def _sc_edge(Np, Ep, layer):
    EPW = Ep // NW
    nchunk = EPW // CE
    assert nchunk % 2 == 0
    RZ = Np // NS

    @functools.partial(
        pl.kernel,
        out_type=jax.ShapeDtypeStruct((NC, Np, HID), jnp.float32),
        mesh=_mesh(),
        scratch_types=[
            pltpu.VMEM((2, CE), jnp.int32), pltpu.VMEM((2, CE), jnp.int32),
            pltpu.VMEM((2, CE, HID), jnp.float32),
            pltpu.VMEM((2, CE, HID), jnp.float32),
            pltpu.VMEM((2, CE, HID), jnp.float32),
            pltpu.VMEM_SHARED((Np, HID), jnp.float32),
            pltpu.SemaphoreType.DMA, pltpu.SemaphoreType.DMA,
            pltpu.SemaphoreType.DMA, pltpu.SemaphoreType.DMA,
        ],
    )
    def k(a_hbm, b_hbm, ii_hbm, jj_hbm, rp_hbm, s_out,
          ii_v, jj_v, av, bv, rv, s_sh, g0, g1, i0, i1):
        cid = lax.axis_index("c")
        sid = lax.axis_index("s")
        w = sid * NC + cid
        base0 = w * EPW
        zero16 = jnp.zeros((16,), jnp.float32)
        sems = (g0, g1)
        isems = (i0, i1)

        def zloop(f, carry):
            r = f // (HID // 16)
            g = (f % (HID // 16)) * 16
            av[0, r, pl.ds(g, 16)] = zero16
            return carry

        lax.fori_loop(0, CE * (HID // 16), zloop, 0)
        for t in range(RZ // CE):
            pltpu.sync_copy(av.at[0], s_sh.at[pl.ds(sid * RZ + t * CE, CE)])
        plsc.subcore_barrier()

        def start_idx(ph, kk):
            b = base0 + kk * CE
            pltpu.async_copy(ii_hbm.at[pl.ds(b, CE)], ii_v.at[ph], isems[ph])
            pltpu.async_copy(jj_hbm.at[pl.ds(b, CE)], jj_v.at[ph], isems[ph])

        def drain_idx(ph):
            dummy = ii_hbm.at[pl.ds(0, CE)]
            pltpu.make_async_copy(dummy, ii_v.at[ph], isems[ph]).wait()
            pltpu.make_async_copy(dummy, jj_v.at[ph], isems[ph]).wait()

        def start(ph, kk):
            pltpu.async_copy(a_hbm.at[ii_v.at[ph]], av.at[ph], sems[ph])
            pltpu.async_copy(b_hbm.at[jj_v.at[ph]], bv.at[ph], sems[ph])
            pltpu.async_copy(rp_hbm.at[layer, pl.ds(base0 + kk * CE, CE)],
                             rv.at[ph], sems[ph])

        def drain(ph):
            dummy = a_hbm.at[pl.ds(0, CE)]
            pltpu.make_async_copy(dummy, av.at[ph], sems[ph]).wait()
            pltpu.make_async_copy(dummy, bv.at[ph], sems[ph]).wait()
            pltpu.make_async_copy(dummy, rv.at[ph], sems[ph]).wait()

        start_idx(0, 0)
        drain_idx(0)
        start(0, 0)
        start_idx(1, 1)

        def body2(t, carry):
            for ph in (0, 1):
                kk = 2 * t + ph
                nxt = 1 - ph

                @pl.when(kk + 1 < nchunk)
                def _():
                    drain_idx(nxt)
                    start(nxt, kk + 1)

                drain(ph)

                def comp(r, cc):
                    for g in range(HID // 16):
                        o = g * 16
                        av[ph, r, pl.ds(o, 16)] = jnp.maximum(
                            av[ph, r, pl.ds(o, 16)] + bv[ph, r, pl.ds(o, 16)]
                            + rv[ph, r, pl.ds(o, 16)], 0.0)
                    return cc

                lax.fori_loop(0, CE, comp, 0)
                pltpu.sync_copy(av.at[ph], s_sh.at[ii_v.at[ph]], add=True)

                @pl.when(kk + 2 < nchunk)
                def _():
                    start_idx(ph, kk + 2)
            return carry

        lax.fori_loop(0, nchunk // 2, body2, 0)
        plsc.subcore_barrier()
        for t in range(RZ // CE):
            off = sid * RZ + t * CE
            pltpu.sync_copy(s_sh.at[pl.ds(off, CE)], s_out.at[cid, pl.ds(off, CE)])

    return k


@functools.lru_cache(maxsize=None)
def _tc_rp(Ep):
    Rb = 2048

    def body(pi_ref, pj_ref, w_ref, b_ref, out_ref):
        diff = pi_ref[...] - pj_ref[...]
        d = jnp.sqrt(jnp.sum(diff * diff, axis=1, keepdims=True))
        centers = (CUTOFF / (RBF - 1)) * lax.broadcasted_iota(
            jnp.int32, (1, RBF), 1).astype(jnp.float32)
        rbf = jnp.exp(-GAMMA * (d - centers) ** 2)
        out_ref[0] = jnp.dot(rbf, w_ref[0],
                             preferred_element_type=jnp.float32) + b_ref[0]

    return pl.pallas_call(
        body,
        grid=(NL, Ep // Rb),
        in_specs=[
            pl.BlockSpec((Rb, 8), lambda l, e: (e, 0)),
            pl.BlockSpec((Rb, 8), lambda l, e: (e, 0)),
            pl.BlockSpec((1, RBF, HID), lambda l, e: (l, 0, 0)),
            pl.BlockSpec((1, 1, HID), lambda l, e: (l, 0, 0)),
        ],
        out_specs=pl.BlockSpec((1, Rb, HID), lambda l, e: (l, e, 0)),
        out_shape=jax.ShapeDtypeStruct((NL, Ep, HID), jnp.float32),
    )


@functools.lru_cache(maxsize=None)
def _tc_embed(Np):
    def body(z_ref, e_ref, wi_ref, wj_ref, h_ref, a_ref, b_ref):
        ids = lax.broadcasted_iota(jnp.int32, (1, 16), 1).astype(jnp.float32)
        oh = (z_ref[...] == ids).astype(jnp.float32)
        h = jnp.dot(oh, e_ref[...], preferred_element_type=jnp.float32)
        h_ref[...] = h
        a_ref[...] = jnp.dot(h, wi_ref[...], preferred_element_type=jnp.float32)
        b_ref[...] = jnp.dot(h, wj_ref[...], preferred_element_type=jnp.float32)

    sds = jax.ShapeDtypeStruct((Np, HID), jnp.float32)
    return pl.pallas_call(
        body,
        grid=(Np // RN,),
        in_specs=[
            pl.BlockSpec((RN, 1), lambda i: (i, 0)),
            pl.BlockSpec((16, HID), lambda i: (0, 0)),
            pl.BlockSpec((HID, HID), lambda i: (0, 0)),
            pl.BlockSpec((HID, HID), lambda i: (0, 0)),
        ],
        out_specs=[pl.BlockSpec((RN, HID), lambda i: (i, 0))] * 3,
        out_shape=[sds, sds, sds],
    )


def _node_update(s_ref, dg_ref, h_ref, w2_ref, b2_ref, u1a_ref, u1b_ref,
                 c1_ref, u2_ref, c2_ref):
    ssum = s_ref[0] + s_ref[1]
    deg = dg_ref[0, :, :1] + dg_ref[1, :, :1]
    aggr = (jnp.dot(ssum, w2_ref[...], preferred_element_type=jnp.float32)
            + deg * b2_ref[...]) / jnp.maximum(deg, 1.0)
    u = jnp.maximum(
        jnp.dot(h_ref[...], u1a_ref[...], preferred_element_type=jnp.float32)
        + jnp.dot(aggr, u1b_ref[...], preferred_element_type=jnp.float32)
        + c1_ref[...], 0.0)
    return jnp.dot(u, u2_ref[...], preferred_element_type=jnp.float32) + c2_ref[...]


_W = pl.BlockSpec((HID, HID), lambda i: (0, 0))
_B = pl.BlockSpec((1, HID), lambda i: (0, 0))
_S = pl.BlockSpec((NC, RN, HID), lambda i: (0, i, 0))
_D = pl.BlockSpec((NC, RN, 16), lambda i: (0, i, 0))
_H = pl.BlockSpec((RN, HID), lambda i: (i, 0))


@functools.lru_cache(maxsize=None)
def _tc_update(Np):
    def body(s_ref, dg_ref, h_ref, w2_ref, b2_ref, u1a_ref, u1b_ref, c1_ref,
             u2_ref, c2_ref, wi_ref, wj_ref, hn_ref, an_ref, bn_ref):
        hn = _node_update(s_ref, dg_ref, h_ref, w2_ref, b2_ref, u1a_ref,
                          u1b_ref, c1_ref, u2_ref, c2_ref)
        hn_ref[...] = hn
        an_ref[...] = jnp.dot(hn, wi_ref[...], preferred_element_type=jnp.float32)
        bn_ref[...] = jnp.dot(hn, wj_ref[...], preferred_element_type=jnp.float32)

    sds = jax.ShapeDtypeStruct((Np, HID), jnp.float32)
    return pl.pallas_call(
        body,
        grid=(Np // RN,),
        in_specs=[_S, _D, _H, _W, _B, _W, _W, _B, _W, _B, _W, _W],
        out_specs=[_H] * 3,
        out_shape=[sds, sds, sds],
    )


@functools.lru_cache(maxsize=None)
def _tc_final(Np, N):
    def body(s_ref, dg_ref, h_ref, w2_ref, b2_ref, u1a_ref, u1b_ref, c1_ref,
             u2_ref, c2_ref, ow1_ref, ob1_ref, ow2_ref, ob2_ref, out_ref):
        pid = pl.program_id(0)
        hn = _node_update(s_ref, dg_ref, h_ref, w2_ref, b2_ref, u1a_ref,
                          u1b_ref, c1_ref, u2_ref, c2_ref)
        hid = jnp.maximum(
            jnp.dot(hn, ow1_ref[...], preferred_element_type=jnp.float32)
            + ob1_ref[...], 0.0)
        eatom = (jnp.dot(hid, ow2_ref[...], preferred_element_type=jnp.float32)
                 + ob2_ref[...])
        rid = pid * RN + lax.broadcasted_iota(jnp.int32, (RN, 1), 0)
        blk = jnp.full((1, HID), jnp.sum(jnp.where(rid < N, eatom, 0.0)))

        @pl.when(pid == 0)
        def _():
            out_ref[...] = blk

        @pl.when(pid != 0)
        def _():
            out_ref[...] += blk

    return pl.pallas_call(
        body,
        grid=(Np // RN,),
        in_specs=[
            _S, _D, _H, _W, _B, _W, _W, _B, _W, _B,
            _W, _B,
            pl.BlockSpec((HID, 1), lambda i: (0, 0)),
            pl.BlockSpec((1, 1), lambda i: (0, 0)),
        ],
        out_specs=pl.BlockSpec((1, HID), lambda i: (0, 0)),
        out_shape=jax.ShapeDtypeStruct((1, HID), jnp.float32),
    )


def kernel(z, pos, edge_index, emb, msg_w1, msg_b1, msg_w2, msg_b2,
           upd_w1, upd_b1, upd_w2, upd_b2, out_w1, out_b1, out_w2, out_b2):
    N = z.shape[0]
    E = edge_index.shape[1]
    Np = -(-N // (NS * C)) * (NS * C)
    Ep = -(-E // (NW * C)) * (NW * C)

    zf = jnp.pad(z.astype(jnp.float32), (0, Np - N)).reshape(Np, 1)
    pos8 = jnp.pad(pos, ((0, Np - N), (0, 5)))
    ii = jnp.pad(edge_index[0].astype(jnp.int32), (0, Ep - E),
                 constant_values=Np - 1)
    jj = jnp.pad(edge_index[1].astype(jnp.int32), (0, Ep - E),
                 constant_values=Np - 1)
    ii64 = ii.reshape(NW, Ep // NW // C, C)
    emb16 = jnp.pad(emb, ((0, 16 - emb.shape[0]), (0, 0)))
    w1i = msg_w1[:, :HID, :]
    w1j = msg_w1[:, HID:2 * HID, :]
    w1r = msg_w1[:, 2 * HID:, :]

    pi8, pj8, dg = _sc_prep(Np, Ep)(pos8, ii64, jj)
    rp = _tc_rp(Ep)(pi8, pj8, w1r, msg_b1.reshape(NL, 1, HID))
    h, a, b = _tc_embed(Np)(zf, emb16, w1i[0], w1j[0])

    for l in range(NL - 1):
        s2 = _sc_edge(Np, Ep, l)(a, b, ii, jj, rp)
        h, a, b = _tc_update(Np)(
            s2, dg, h, msg_w2[l], msg_b2[l].reshape(1, HID),
            upd_w1[l, :HID], upd_w1[l, HID:], upd_b1[l].reshape(1, HID),
            upd_w2[l], upd_b2[l].reshape(1, HID), w1i[l + 1], w1j[l + 1])

    s2 = _sc_edge(Np, Ep, NL - 1)(a, b, ii, jj, rp)
    energy = _tc_final(Np, N)(
        s2, dg, h, msg_w2[NL - 1], msg_b2[NL - 1].reshape(1, HID),
        upd_w1[NL - 1, :HID], upd_w1[NL - 1, HID:],
        upd_b1[NL - 1].reshape(1, HID), upd_w2[NL - 1],
        upd_b2[NL - 1].reshape(1, HID), out_w1, out_b1.reshape(1, HID),
        out_w2, out_b2.reshape(1, 1))
    return energy[0, 0]


# speedup vs baseline: 3.8113x; 1.0639x over previous
import functools

import jax
import jax.numpy as jnp
from jax import lax
from jax.experimental import pallas as pl
from jax.experimental.pallas import tpu as pltpu
from jax.experimental.pallas import tpu_sc as plsc

HID = 128
RBF = 32
NL = 4
CUTOFF = 10.0
GAMMA = RBF / CUTOFF
NC = 2
NS = 16
NW = NC * NS
C = 64
RN = 1024


def _mesh():
    return plsc.VectorSubcoreMesh(core_axis_name="c", subcore_axis_name="s",
                                  num_cores=NC, num_subcores=NS)


@functools.lru_cache(maxsize=None)
def _sc_prep(Np, Ep):
    EPW = Ep // NW
    nchunk = EPW // C
    RZ = Np // NS

    @functools.partial(
        pl.kernel,
        out_type=(jax.ShapeDtypeStruct((Ep, 8), jnp.float32),
                  jax.ShapeDtypeStruct((Ep, 8), jnp.float32),
                  jax.ShapeDtypeStruct((NC, Np, 16), jnp.float32)),
        mesh=_mesh(),
        scratch_types=[
            pltpu.VMEM((EPW // C, C), jnp.int32),
            pltpu.VMEM((EPW,), jnp.int32),
            pltpu.VMEM((C, 8), jnp.float32), pltpu.VMEM((C, 8), jnp.float32),
            pltpu.VMEM((C, 16), jnp.float32),
            pltpu.VMEM_SHARED((Np, 16), jnp.float32),
            pltpu.SemaphoreType.DMA, pltpu.SemaphoreType.DMA,
        ],
        compiler_params=pltpu.CompilerParams(use_tc_tiling_on_sc=False),
    )
    def k(pos_hbm, ii2_hbm, jj_hbm, pi_out, pj_out, dg_out,
          ii_loc, jj_loc, pi_v, pj_v, ov, d_sh, s1, s2):
        cid = lax.axis_index("c")
        sid = lax.axis_index("s")
        w = sid * NC + cid
        base0 = w * EPW
        zero16 = jnp.zeros((16,), jnp.float32)
        pltpu.sync_copy(ii2_hbm.at[w], ii_loc)
        pltpu.sync_copy(jj_hbm.at[pl.ds(base0, EPW)], jj_loc)

        def zloop(r, carry):
            ov[r, pl.ds(0, 16)] = zero16
            return carry

        lax.fori_loop(0, C, zloop, 0)
        for t in range(RZ // C):
            pltpu.sync_copy(ov, d_sh.at[pl.ds(sid * RZ + t * C, C)])
        plsc.subcore_barrier()

        dvec = jnp.where(lax.iota(jnp.int32, 16) == 0, 1.0, 0.0).astype(
            jnp.float32)

        def dloop(r, carry):
            ov[r, pl.ds(0, 16)] = dvec
            return carry

        lax.fori_loop(0, C, dloop, 0)

        def chunk(t, carry):
            b = base0 + t * C
            c1 = pltpu.async_copy(pos_hbm.at[ii_loc.at[t]], pi_v, s1)
            c2 = pltpu.async_copy(pos_hbm.at[jj_loc.at[pl.ds(t * C, C)]],
                                  pj_v, s2)
            c1.wait()
            c2.wait()
            pltpu.sync_copy(pi_v, pi_out.at[pl.ds(b, C)])
            pltpu.sync_copy(pj_v, pj_out.at[pl.ds(b, C)])
            pltpu.sync_copy(ov, d_sh.at[ii_loc.at[t]], add=True)
            return carry

        lax.fori_loop(0, nchunk, chunk, 0)
        plsc.subcore_barrier()
        for t in range(RZ // C):
            off = sid * RZ + t * C
            pltpu.sync_copy(d_sh.at[pl.ds(off, C)], dg_out.at[cid, pl.ds(off, C)])

    return k


CE = 32


@functools.lru_cache(maxsize=None)
def _sc_edge(Np, Ep, layer):
    EPW = Ep // NW
    nchunk = EPW // CE
    assert nchunk % 2 == 0
    RZ = Np // NS

    @functools.partial(
        pl.kernel,
        out_type=jax.ShapeDtypeStruct((NC, Np, HID), jnp.float32),
        mesh=_mesh(),
        scratch_types=[
            pltpu.VMEM((2, CE), jnp.int32), pltpu.VMEM((2, CE), jnp.int32),
            pltpu.VMEM((2, CE, HID), jnp.float32),
            pltpu.VMEM((2, CE, HID), jnp.float32),
            pltpu.VMEM((2, CE, HID), jnp.float32),
            pltpu.VMEM_SHARED((Np, HID), jnp.float32),
            pltpu.SemaphoreType.DMA, pltpu.SemaphoreType.DMA,
            pltpu.SemaphoreType.DMA, pltpu.SemaphoreType.DMA,
        ],
    )
    def k(a_hbm, b_hbm, ii_hbm, jj_hbm, rp_hbm, s_out,
          ii_v, jj_v, av, bv, rv, s_sh, g0, g1, i0, i1):
        cid = lax.axis_index("c")
        sid = lax.axis_index("s")
        w = sid * NC + cid
        base0 = w * EPW
        zero16 = jnp.zeros((16,), jnp.float32)
        sems = (g0, g1)
        isems = (i0, i1)

        def zloop(f, carry):
            r = f // (HID // 16)
            g = (f % (HID // 16)) * 16
            av[0, r, pl.ds(g, 16)] = zero16
            return carry

        lax.fori_loop(0, CE * (HID // 16), zloop, 0)
        for t in range(RZ // CE):
            pltpu.sync_copy(av.at[0], s_sh.at[pl.ds(sid * RZ + t * CE, CE)])
        plsc.subcore_barrier()

        def start_idx(ph, kk):
            b = base0 + kk * CE
            pltpu.async_copy(ii_hbm.at[pl.ds(b, CE)], ii_v.at[ph], isems[ph])
            pltpu.async_copy(jj_hbm.at[pl.ds(b, CE)], jj_v.at[ph], isems[ph])

        def drain_idx(ph):
            dummy = ii_hbm.at[pl.ds(0, CE)]
            pltpu.make_async_copy(dummy, ii_v.at[ph], isems[ph]).wait()
            pltpu.make_async_copy(dummy, jj_v.at[ph], isems[ph]).wait()

        def start(ph, kk):
            pltpu.async_copy(a_hbm.at[ii_v.at[ph]], av.at[ph], sems[ph])
            pltpu.async_copy(b_hbm.at[jj_v.at[ph]], bv.at[ph], sems[ph])
            pltpu.async_copy(rp_hbm.at[layer, pl.ds(base0 + kk * CE, CE)],
                             rv.at[ph], sems[ph])

        def drain(ph):
            dummy = a_hbm.at[pl.ds(0, CE)]
            pltpu.make_async_copy(dummy, av.at[ph], sems[ph]).wait()
            pltpu.make_async_copy(dummy, bv.at[ph], sems[ph]).wait()
            pltpu.make_async_copy(dummy, rv.at[ph], sems[ph]).wait()

        start_idx(0, 0)
        drain_idx(0)
        start(0, 0)
        start_idx(1, 1)

        def body2(t, carry):
            for ph in (0, 1):
                kk = 2 * t + ph
                nxt = 1 - ph

                @pl.when(kk + 1 < nchunk)
                def _():
                    drain_idx(nxt)
                    start(nxt, kk + 1)

                drain(ph)

                def comp(r, cc):
                    for g in range(HID // 16):
                        o = g * 16
                        av[ph, r, pl.ds(o, 16)] = jnp.maximum(
                            av[ph, r, pl.ds(o, 16)] + bv[ph, r, pl.ds(o, 16)]
                            + rv[ph, r, pl.ds(o, 16)], 0.0)
                    return cc

                lax.fori_loop(0, CE, comp, 0)
                pltpu.sync_copy(av.at[ph], s_sh.at[ii_v.at[ph]], add=True)

                @pl.when(kk + 2 < nchunk)
                def _():
                    start_idx(ph, kk + 2)
            return carry

        lax.fori_loop(0, nchunk // 2, body2, 0)
        plsc.subcore_barrier()
        for t in range(RZ // CE):
            off = sid * RZ + t * CE
            pltpu.sync_copy(s_sh.at[pl.ds(off, CE)], s_out.at[cid, pl.ds(off, CE)])

    return k


@functools.lru_cache(maxsize=None)
def _tc_rp(Ep):
    Rb = 2048

    def body(pi_ref, pj_ref, w_ref, b_ref, out_ref, rbf_ref):
        l = pl.program_id(1)

        @pl.when(l == 0)
        def _():
            diff = pi_ref[...] - pj_ref[...]
            d = jnp.sqrt(jnp.sum(diff * diff, axis=1, keepdims=True))
            centers = (CUTOFF / (RBF - 1)) * lax.broadcasted_iota(
                jnp.int32, (1, RBF), 1).astype(jnp.float32)
            rbf_ref[...] = jnp.exp(-GAMMA * (d - centers) ** 2)

        out_ref[0] = jnp.dot(rbf_ref[...], w_ref[0],
                             preferred_element_type=jnp.float32) + b_ref[0]

    return pl.pallas_call(
        body,
        grid=(Ep // Rb, NL),
        in_specs=[
            pl.BlockSpec((Rb, 8), lambda e, l: (e, 0)),
            pl.BlockSpec((Rb, 8), lambda e, l: (e, 0)),
            pl.BlockSpec((1, RBF, HID), lambda e, l: (l, 0, 0)),
            pl.BlockSpec((1, 1, HID), lambda e, l: (l, 0, 0)),
        ],
        out_specs=pl.BlockSpec((1, Rb, HID), lambda e, l: (l, e, 0)),
        out_shape=jax.ShapeDtypeStruct((NL, Ep, HID), jnp.float32),
        scratch_shapes=[pltpu.VMEM((Rb, RBF), jnp.float32)],
    )


@functools.lru_cache(maxsize=None)
def _tc_embed(Np):
    def body(z_ref, e_ref, wi_ref, wj_ref, h_ref, a_ref, b_ref):
        ids = lax.broadcasted_iota(jnp.int32, (1, 16), 1).astype(jnp.float32)
        oh = (z_ref[...] == ids).astype(jnp.float32)
        h = jnp.dot(oh, e_ref[...], preferred_element_type=jnp.float32)
        h_ref[...] = h
        a_ref[...] = jnp.dot(h, wi_ref[...], preferred_element_type=jnp.float32)
        b_ref[...] = jnp.dot(h, wj_ref[...], preferred_element_type=jnp.float32)

    sds = jax.ShapeDtypeStruct((Np, HID), jnp.float32)
    return pl.pallas_call(
        body,
        grid=(Np // RN,),
        in_specs=[
            pl.BlockSpec((RN, 1), lambda i: (i, 0)),
            pl.BlockSpec((16, HID), lambda i: (0, 0)),
            pl.BlockSpec((HID, HID), lambda i: (0, 0)),
            pl.BlockSpec((HID, HID), lambda i: (0, 0)),
        ],
        out_specs=[pl.BlockSpec((RN, HID), lambda i: (i, 0))] * 3,
        out_shape=[sds, sds, sds],
    )


def _node_update(s_ref, dg_ref, h_ref, w2_ref, b2_ref, u1a_ref, u1b_ref,
                 c1_ref, u2_ref, c2_ref):
    ssum = s_ref[0] + s_ref[1]
    deg = dg_ref[0, :, :1] + dg_ref[1, :, :1]
    aggr = (jnp.dot(ssum, w2_ref[...], preferred_element_type=jnp.float32)
            + deg * b2_ref[...]) / jnp.maximum(deg, 1.0)
    u = jnp.maximum(
        jnp.dot(h_ref[...], u1a_ref[...], preferred_element_type=jnp.float32)
        + jnp.dot(aggr, u1b_ref[...], preferred_element_type=jnp.float32)
        + c1_ref[...], 0.0)
    return jnp.dot(u, u2_ref[...], preferred_element_type=jnp.float32) + c2_ref[...]


_W = pl.BlockSpec((HID, HID), lambda i: (0, 0))
_B = pl.BlockSpec((1, HID), lambda i: (0, 0))
_S = pl.BlockSpec((NC, RN, HID), lambda i: (0, i, 0))
_D = pl.BlockSpec((NC, RN, 16), lambda i: (0, i, 0))
_H = pl.BlockSpec((RN, HID), lambda i: (i, 0))


@functools.lru_cache(maxsize=None)
def _tc_update(Np):
    def body(s_ref, dg_ref, h_ref, w2_ref, b2_ref, u1a_ref, u1b_ref, c1_ref,
             u2_ref, c2_ref, wi_ref, wj_ref, hn_ref, an_ref, bn_ref):
        hn = _node_update(s_ref, dg_ref, h_ref, w2_ref, b2_ref, u1a_ref,
                          u1b_ref, c1_ref, u2_ref, c2_ref)
        hn_ref[...] = hn
        an_ref[...] = jnp.dot(hn, wi_ref[...], preferred_element_type=jnp.float32)
        bn_ref[...] = jnp.dot(hn, wj_ref[...], preferred_element_type=jnp.float32)

    sds = jax.ShapeDtypeStruct((Np, HID), jnp.float32)
    return pl.pallas_call(
        body,
        grid=(Np // RN,),
        in_specs=[_S, _D, _H, _W, _B, _W, _W, _B, _W, _B, _W, _W],
        out_specs=[_H] * 3,
        out_shape=[sds, sds, sds],
    )


@functools.lru_cache(maxsize=None)
def _tc_final(Np, N):
    def body(s_ref, dg_ref, h_ref, w2_ref, b2_ref, u1a_ref, u1b_ref, c1_ref,
             u2_ref, c2_ref, ow1_ref, ob1_ref, ow2_ref, ob2_ref, out_ref):
        pid = pl.program_id(0)
        hn = _node_update(s_ref, dg_ref, h_ref, w2_ref, b2_ref, u1a_ref,
                          u1b_ref, c1_ref, u2_ref, c2_ref)
        hid = jnp.maximum(
            jnp.dot(hn, ow1_ref[...], preferred_element_type=jnp.float32)
            + ob1_ref[...], 0.0)
        eatom = (jnp.dot(hid, ow2_ref[...], preferred_element_type=jnp.float32)
                 + ob2_ref[...])
        rid = pid * RN + lax.broadcasted_iota(jnp.int32, (RN, 1), 0)
        blk = jnp.full((1, HID), jnp.sum(jnp.where(rid < N, eatom, 0.0)))

        @pl.when(pid == 0)
        def _():
            out_ref[...] = blk

        @pl.when(pid != 0)
        def _():
            out_ref[...] += blk

    return pl.pallas_call(
        body,
        grid=(Np // RN,),
        in_specs=[
            _S, _D, _H, _W, _B, _W, _W, _B, _W, _B,
            _W, _B,
            pl.BlockSpec((HID, 1), lambda i: (0, 0)),
            pl.BlockSpec((1, 1), lambda i: (0, 0)),
        ],
        out_specs=pl.BlockSpec((1, HID), lambda i: (0, 0)),
        out_shape=jax.ShapeDtypeStruct((1, HID), jnp.float32),
    )


def kernel(z, pos, edge_index, emb, msg_w1, msg_b1, msg_w2, msg_b2,
           upd_w1, upd_b1, upd_w2, upd_b2, out_w1, out_b1, out_w2, out_b2):
    N = z.shape[0]
    E = edge_index.shape[1]
    Np = -(-N // (NS * C)) * (NS * C)
    Ep = -(-E // (NW * C)) * (NW * C)

    zf = jnp.pad(z.astype(jnp.float32), (0, Np - N)).reshape(Np, 1)
    pos8 = jnp.pad(pos, ((0, Np - N), (0, 5)))
    ii = jnp.pad(edge_index[0].astype(jnp.int32), (0, Ep - E),
                 constant_values=Np - 1)
    jj = jnp.pad(edge_index[1].astype(jnp.int32), (0, Ep - E),
                 constant_values=Np - 1)
    ii64 = ii.reshape(NW, Ep // NW // C, C)
    emb16 = jnp.pad(emb, ((0, 16 - emb.shape[0]), (0, 0)))
    w1i = msg_w1[:, :HID, :]
    w1j = msg_w1[:, HID:2 * HID, :]
    w1r = msg_w1[:, 2 * HID:, :]

    pi8, pj8, dg = _sc_prep(Np, Ep)(pos8, ii64, jj)
    rp = _tc_rp(Ep)(pi8, pj8, w1r, msg_b1.reshape(NL, 1, HID))
    h, a, b = _tc_embed(Np)(zf, emb16, w1i[0], w1j[0])

    for l in range(NL - 1):
        s2 = _sc_edge(Np, Ep, l)(a, b, ii, jj, rp)
        h, a, b = _tc_update(Np)(
            s2, dg, h, msg_w2[l], msg_b2[l].reshape(1, HID),
            upd_w1[l, :HID], upd_w1[l, HID:], upd_b1[l].reshape(1, HID),
            upd_w2[l], upd_b2[l].reshape(1, HID), w1i[l + 1], w1j[l + 1])

    s2 = _sc_edge(Np, Ep, NL - 1)(a, b, ii, jj, rp)
    energy = _tc_final(Np, N)(
        s2, dg, h, msg_w2[NL - 1], msg_b2[NL - 1].reshape(1, HID),
        upd_w1[NL - 1, :HID], upd_w1[NL - 1, HID:],
        upd_b1[NL - 1].reshape(1, HID), upd_w2[NL - 1],
        upd_b2[NL - 1].reshape(1, HID), out_w1, out_b1.reshape(1, HID),
        out_w2, out_b2.reshape(1, 1))
    return energy[0, 0]
